# baseline (speedup 1.0000x reference)
import functools

import jax
import jax.numpy as jnp
from jax import lax
from jax.experimental import pallas as pl
from jax.experimental.pallas import tpu as pltpu
from jax.experimental.pallas import tpu_sc as plsc

N = 8192
C_IN = 128
C_OUT = 256
K = 16
M = N // 4
EPS = 1e-5
BIGI = 2**30


_FR, _FC = 64, 128
_KT = 128
_CHR = 512
INF = float("inf")


def _geo_body(px_ref, py_ref, pz_ref, pxc_ref, pyc_ref, pzc_ref,
              npx_ref, npy_ref, npz_ref, knn_ref, d2_ref):
    i = pl.program_id(0)

    @pl.when(i == 0)
    def _fps():
        rows = lax.broadcasted_iota(jnp.int32, (_FR, _FC), 0)
        cols = lax.broadcasted_iota(jnp.int32, (_FR, _FC), 1)
        lin = rows * _FC + cols
        l128 = lax.broadcasted_iota(jnp.int32, (1, _FC), 1)
        px = px_ref[...]
        py = py_ref[...]
        pz = pz_ref[...]
        qx0 = px_ref[0:1, 0:1]
        qy0 = py_ref[0:1, 0:1]
        qz0 = pz_ref[0:1, 0:1]
        zrow = jnp.zeros((1, _FC), jnp.float32)
        rax0 = jnp.where(l128 == 0, qx0, zrow)
        ray0 = jnp.where(l128 == 0, qy0, zrow)
        raz0 = jnp.where(l128 == 0, qz0, zrow)

        def tree_min(v):
            v = jnp.minimum(v[0:32], v[32:64])
            v = jnp.minimum(v[0:16], v[16:32])
            v = jnp.minimum(v[0:8], v[8:16])
            v = jnp.min(v, axis=0, keepdims=True)
            return jnp.min(v, axis=1, keepdims=True)

        def step(j, carry):
            dists, qx, qy, qz, rax, ray, raz = carry
            dx = px - qx
            dy = py - qy
            dz = pz - qz
            d = dx * dx + dy * dy + dz * dz
            dists = jnp.minimum(dists, d)
            a = jnp.maximum(dists[0:32], dists[32:64])
            a = jnp.maximum(a[0:16], a[16:32])
            a = jnp.maximum(a[0:8], a[8:16])
            a = jnp.max(a, axis=0, keepdims=True)
            mx = jnp.max(a, axis=1, keepdims=True)
            cand = jnp.where(dists == mx, lin, BIGI)
            imin = tree_min(cand)
            nx = tree_min(jnp.where(lin == imin, px, INF))
            ny = tree_min(jnp.where(lin == imin, py, INF))
            nz = tree_min(jnp.where(lin == imin, pz, INF))
            r = j // _FC
            l = j % _FC
            rax = jnp.where(l128 == l, nx, rax)
            ray = jnp.where(l128 == l, ny, ray)
            raz = jnp.where(l128 == l, nz, raz)

            @pl.when(l == _FC - 1)
            def _():
                npx_ref[pl.ds(r, 1), :] = rax
                npy_ref[pl.ds(r, 1), :] = ray
                npz_ref[pl.ds(r, 1), :] = raz

            keep = l < _FC - 1
            rax = jnp.where(keep, rax, 0.0)
            ray = jnp.where(keep, ray, 0.0)
            raz = jnp.where(keep, raz, 0.0)
            return (dists, nx, ny, nz, rax, ray, raz)

        init = (jnp.full((_FR, _FC), 1e10, jnp.float32), qx0, qy0, qz0,
                rax0, ray0, raz0)
        lax.fori_loop(1, M, step, init)

    @pl.when(i > 0)
    def _knn():
        t = i - 1
        cx = npx_ref[pl.ds(t, 1), :]
        cy = npy_ref[pl.ds(t, 1), :]
        cz = npz_ref[pl.ds(t, 1), :]
        for c in range(N // _CHR):
            s = c * _CHR
            dx = pxc_ref[s:s + _CHR, :] - cx
            dy = pyc_ref[s:s + _CHR, :] - cy
            dz = pzc_ref[s:s + _CHR, :] - cz
            d2_ref[s:s + _CHR, :] = dx * dx + dy * dy + dz * dz
        iprev = jnp.full((1, _KT), -1, jnp.int32)
        for k in range(K):
            vbest = jnp.full((1, _KT), INF, jnp.float32)
            for c in range(N // _CHR):
                s = c * _CHR
                ri = lax.broadcasted_iota(jnp.int32, (_CHR, _KT), 0) + s
                blk = d2_ref[s:s + _CHR, :]
                if k > 0:
                    blk = jnp.where(ri == iprev, INF, blk)
                    d2_ref[s:s + _CHR, :] = blk
                vbest = jnp.minimum(vbest,
                                    jnp.min(blk, axis=0, keepdims=True))
            ibest = jnp.full((1, _KT), BIGI, jnp.int32)
            for c in range(N // _CHR):
                s = c * _CHR
                ri = lax.broadcasted_iota(jnp.int32, (_CHR, _KT), 0) + s
                blk = d2_ref[s:s + _CHR, :]
                cand = jnp.min(jnp.where(blk == vbest, ri, BIGI),
                               axis=0, keepdims=True)
                ibest = jnp.minimum(ibest, cand)
            knn_ref[k:k + 1, :] = ibest
            iprev = ibest


def _run_geo(p):
    px = p[:, 0].reshape(_FR, _FC)
    py = p[:, 1].reshape(_FR, _FC)
    pz = p[:, 2].reshape(_FR, _FC)
    cmap = lambda i: (0, 0)
    return pl.pallas_call(
        _geo_body,
        grid=(M // _KT + 1,),
        in_specs=[
            pl.BlockSpec((_FR, _FC), cmap),
            pl.BlockSpec((_FR, _FC), cmap),
            pl.BlockSpec((_FR, _FC), cmap),
            pl.BlockSpec((N, 1), cmap),
            pl.BlockSpec((N, 1), cmap),
            pl.BlockSpec((N, 1), cmap),
        ],
        out_specs=[
            pl.BlockSpec((M // _FC, _FC), cmap),
            pl.BlockSpec((M // _FC, _FC), cmap),
            pl.BlockSpec((M // _FC, _FC), cmap),
            pl.BlockSpec((K, _KT), lambda i: (0, jnp.maximum(i - 1, 0))),
        ],
        out_shape=[
            jax.ShapeDtypeStruct((M // _FC, _FC), jnp.float32),
            jax.ShapeDtypeStruct((M // _FC, _FC), jnp.float32),
            jax.ShapeDtypeStruct((M // _FC, _FC), jnp.float32),
            jax.ShapeDtypeStruct((K, M), jnp.int32),
        ],
        scratch_shapes=[pltpu.VMEM((N, _KT), jnp.float32)],
    )(px, py, pz, p[:, 0:1], p[:, 1:2], p[:, 2:3])


_MMB = 512
_NMM = N // _MMB


def _mm_body(x_ref, p_ref, w2_ref, wsp_ref, a_ref, g_ref, b_ref,
             y_ref, u_ref, h2_ref, acc_ref):
    ph = pl.program_id(0)
    i = pl.program_id(1)

    @pl.when(ph == 0)
    def _():
        xb = x_ref[...]
        h2 = jnp.dot(xb, w2_ref[...], preferred_element_type=jnp.float32)
        u = (jnp.dot(xb, wsp_ref[...], preferred_element_type=jnp.float32)
             + jnp.dot(p_ref[...], a_ref[...],
                       preferred_element_type=jnp.float32))
        h2_ref[pl.ds(i * _MMB, _MMB), :] = h2
        u_ref[...] = u
        s1 = jnp.sum(h2, axis=0, keepdims=True)
        s2 = jnp.sum(h2 * h2, axis=0, keepdims=True)

        @pl.when(i == 0)
        def _():
            acc_ref[0:1, :] = s1
            acc_ref[1:2, :] = s2

        @pl.when(i > 0)
        def _():
            acc_ref[0:1, :] = acc_ref[0:1, :] + s1
            acc_ref[1:2, :] = acc_ref[1:2, :] + s2

    @pl.when(ph == 1)
    def _():
        mu = acc_ref[0:1, :] / float(N)
        var = acc_ref[1:2, :] / float(N) - mu * mu
        y = (g_ref[...] * (h2_ref[pl.ds(i * _MMB, _MMB), :] - mu) / jnp.sqrt(var + EPS)
             + b_ref[...])
        y_ref[...] = jnp.maximum(y, 0.0)


def _run_mm(x, p, W2, Ws1p, A, g2, b2):
    return pl.pallas_call(
        _mm_body,
        grid=(2, _NMM),
        in_specs=[
            pl.BlockSpec((_MMB, C_IN), lambda p_, i: (i, 0)),
            pl.BlockSpec((_MMB, 3), lambda p_, i: (i, 0)),
            pl.BlockSpec((C_IN, C_OUT), lambda p_, i: (0, 0)),
            pl.BlockSpec((C_IN, C_IN), lambda p_, i: (0, 0)),
            pl.BlockSpec((3, C_IN), lambda p_, i: (0, 0)),
            pl.BlockSpec((1, C_OUT), lambda p_, i: (0, 0)),
            pl.BlockSpec((1, C_OUT), lambda p_, i: (0, 0)),
        ],
        out_specs=[
            pl.BlockSpec((_MMB, C_OUT),
                         lambda p_, i: (jnp.where(p_ == 1, i, 0), 0)),
            pl.BlockSpec((_MMB, C_IN),
                         lambda p_, i: (jnp.where(p_ == 0, i, _NMM - 1), 0)),
        ],
        out_shape=[
            jax.ShapeDtypeStruct((N, C_OUT), jnp.float32),
            jax.ShapeDtypeStruct((N, C_IN), jnp.float32),
        ],
        scratch_shapes=[
            pltpu.VMEM((N, C_OUT), jnp.float32),
            pltpu.VMEM((2, C_OUT), jnp.float32),
        ],
    )(x, p, W2, Ws1p, A, g2.reshape(1, C_OUT), b2.reshape(1, C_OUT))


_NC, _NS = 2, 16
_NW = _NC * _NS
_RW = (M * K) // _NW
_GC = 256


def _sc_gather_body(v_hbm, knn_hbm, vg_hbm, idx0, idx1, idx2, idx3,
                    buf0, buf1, sem0, sem1):
    wid = lax.axis_index("s") * _NC + lax.axis_index("c")
    base = wid * _RW
    idxs = (idx0, idx1, idx2, idx3)
    for c in range(_RW // _GC):
        pltpu.sync_copy(knn_hbm.at[pl.ds(base + c * _GC, _GC)], idxs[c])
    bufs = (buf0, buf1)
    sems = (sem0, sem1)
    cps = [None, None]
    cps[0] = pltpu.async_copy(v_hbm.at[idx0], buf0, sem0)
    for c in range(_RW // _GC):
        if c + 1 < _RW // _GC:
            cps[(c + 1) % 2] = pltpu.async_copy(
                v_hbm.at[idxs[c + 1]], bufs[(c + 1) % 2], sems[(c + 1) % 2])
        cps[c % 2].wait()
        pltpu.sync_copy(bufs[c % 2], vg_hbm.at[pl.ds(base + c * _GC, _GC)])


def _run_sc_gather(v, knn):
    mesh = plsc.VectorSubcoreMesh(core_axis_name="c", subcore_axis_name="s")
    f = pl.kernel(
        _sc_gather_body,
        out_type=jax.ShapeDtypeStruct((M * K, C_IN), jnp.float32),
        mesh=mesh,
        scratch_types=[
            pltpu.VMEM((_GC,), jnp.int32),
            pltpu.VMEM((_GC,), jnp.int32),
            pltpu.VMEM((_GC,), jnp.int32),
            pltpu.VMEM((_GC,), jnp.int32),
            pltpu.VMEM((_GC, C_IN), jnp.float32),
            pltpu.VMEM((_GC, C_IN), jnp.float32),
            pltpu.SemaphoreType.DMA,
            pltpu.SemaphoreType.DMA,
        ],
    )
    return f(v, knn.reshape(M * K))


_PB = 2048
_NPB = (M * K) // _PB


def _shr_body(vg_ref, np_ref, a_ref, g_ref, b_ref, w_ref, bs_ref,
              prob_ref, pre_ref, acc_ref):
    ph = pl.program_id(0)
    i = pl.program_id(1)
    nm = _PB // K
    r16 = lax.broadcasted_iota(jnp.int32, (_PB, nm), 0) // K
    c16 = lax.broadcasted_iota(jnp.int32, (_PB, nm), 1)
    e01 = jnp.where(r16 == c16, 1.0, 0.0)

    @pl.when(ph == 0)
    def _():
        w = jnp.dot(np_ref[...], a_ref[...],
                    preferred_element_type=jnp.float32)
        w_exp = jnp.dot(e01, w, preferred_element_type=jnp.float32)
        pre = vg_ref[...] - w_exp
        pre_ref[pl.ds(i * _PB, _PB), :] = pre
        s1 = jnp.sum(pre, axis=0, keepdims=True)
        s2 = jnp.sum(pre * pre, axis=0, keepdims=True)

        @pl.when(i == 0)
        def _():
            acc_ref[0:1, :] = s1
            acc_ref[1:2, :] = s2

        @pl.when(i > 0)
        def _():
            acc_ref[0:1, :] = acc_ref[0:1, :] + s1
            acc_ref[1:2, :] = acc_ref[1:2, :] + s2

    @pl.when(ph == 1)
    def _():
        n_rows = float(M * K)
        mu = acc_ref[0:1, :] / n_rows
        var = acc_ref[1:2, :] / n_rows - mu * mu
        h = (g_ref[...] * (pre_ref[pl.ds(i * _PB, _PB), :] - mu)
             / jnp.sqrt(var + EPS) + b_ref[...])
        h = jnp.maximum(h, 0.0)
        s = jnp.sum(h * w_ref[...], axis=1, keepdims=True) + bs_ref[0, 0]
        ex = jnp.exp(s)
        seg = jnp.dot(e01.T, ex, preferred_element_type=jnp.float32)
        den = jnp.dot(e01, seg, preferred_element_type=jnp.float32)
        prob_ref[...] = ex / den


def _run_shr(vg, n_p, A, gs1, bs1, Ws2, bs2):
    return pl.pallas_call(
        _shr_body,
        grid=(2, _NPB),
        in_specs=[
            pl.BlockSpec((_PB, C_IN), lambda p_, i: (i, 0)),
            pl.BlockSpec((_PB // K, 3), lambda p_, i: (i, 0)),
            pl.BlockSpec((3, C_IN), lambda p_, i: (0, 0)),
            pl.BlockSpec((1, C_IN), lambda p_, i: (0, 0)),
            pl.BlockSpec((1, C_IN), lambda p_, i: (0, 0)),
            pl.BlockSpec((1, C_IN), lambda p_, i: (0, 0)),
            pl.BlockSpec((1, 1), lambda p_, i: (0, 0),
                         memory_space=pltpu.SMEM),
        ],
        out_specs=pl.BlockSpec(
            (_PB, 1), lambda p_, i: (jnp.where(p_ == 1, i, 0), 0)),
        out_shape=jax.ShapeDtypeStruct((M * K, 1), jnp.float32),
        scratch_shapes=[
            pltpu.VMEM((M * K, C_IN), jnp.float32),
            pltpu.VMEM((2, C_IN), jnp.float32),
        ],
    )(vg, n_p, A, gs1.reshape(1, C_IN), bs1.reshape(1, C_IN),
      Ws2.reshape(1, C_IN), bs2.reshape(1, 1))


_MW = M // _NW
_GM = 4


def _sc_wsum_body(y_hbm, knn_hbm, prob_hbm, out_hbm, idxv, probv, buf0, buf1,
                  outb, sem0, sem1):
    wid = lax.axis_index("s") * _NC + lax.axis_index("c")
    base = wid * _MW * K
    pltpu.sync_copy(knn_hbm.at[pl.ds(base, _MW * K)], idxv)
    pltpu.sync_copy(prob_hbm.at[pl.ds(base, _MW * K)], probv)
    bufs = (buf0, buf1)
    sems = (sem0, sem1)
    dnums = lax.GatherDimensionNumbers(
        offset_dims=(), collapsed_slice_dims=(0,), start_index_map=(0,))

    def one_m(ml, buf):
        pm = probv[pl.ds(ml * K, K)]
        accs = [jnp.zeros((16,), jnp.float32) for _ in range(C_OUT // 16)]
        for k in range(K):
            ik = jnp.zeros((K, 1), jnp.int32) + k
            pk = lax.gather(pm, ik, dnums, (1,),
                            mode=lax.GatherScatterMode.PROMISE_IN_BOUNDS)
            for r in range(C_OUT // 16):
                accs[r] = accs[r] + pk * buf[k, pl.ds(r * 16, 16)]
        for r in range(C_OUT // 16):
            outb[ml, pl.ds(r * 16, 16)] = accs[r]

    def g_body(g, _):
        cps = []
        for j in range(2):
            iv = idxv[pl.ds((g * 2 + j) * K, K)]
            cps.append(pltpu.async_copy(y_hbm.at[iv], bufs[j], sems[j]))
        for j in range(2):
            cps[j].wait()
            one_m(g * 2 + j, bufs[j])
        return 0

    lax.fori_loop(0, _MW // 2, g_body, 0)
    pltpu.sync_copy(outb, out_hbm.at[pl.ds(wid * _MW, _MW)])


def _run_sc_wsum(y, knn, prob):
    mesh = plsc.VectorSubcoreMesh(core_axis_name="c", subcore_axis_name="s")
    f = pl.kernel(
        _sc_wsum_body,
        out_type=jax.ShapeDtypeStruct((M, C_OUT), jnp.float32),
        mesh=mesh,
        scratch_types=[
            pltpu.VMEM((_MW * K,), jnp.int32),
            pltpu.VMEM((_MW * K,), jnp.float32),
            pltpu.VMEM((K, C_OUT), jnp.float32),
            pltpu.VMEM((K, C_OUT), jnp.float32),
            pltpu.VMEM((_MW, C_OUT), jnp.float32),
            pltpu.SemaphoreType.DMA,
            pltpu.SemaphoreType.DMA,
        ],
    )
    return f(y, knn.reshape(M * K), prob.reshape(M * K))


def kernel(p, x, o, W2, g2, b2, Ws1, gs1, bs1, Ws2, bs2):
    npx, npy, npz, knnT = _run_geo(p)
    n_p = jnp.stack([npx.reshape(M), npy.reshape(M), npz.reshape(M)], axis=1)
    knn = knnT.T.reshape(M, K)
    y, v = _run_mm(x, p, W2, Ws1[3:], Ws1[:3], g2, b2)
    vg = _run_sc_gather(v, knn)
    prob = _run_shr(vg, n_p, Ws1[:3], gs1, bs1, Ws2, bs2)
    y_out = _run_sc_wsum(y, knn, prob)
    n_o = jnp.array([M], dtype=jnp.int32)
    return (n_p, y_out, n_o)


# speedup vs baseline: 1.1543x; 1.1543x over previous
import functools

import jax
import jax.numpy as jnp
from jax import lax
from jax.experimental import pallas as pl
from jax.experimental.pallas import tpu as pltpu
from jax.experimental.pallas import tpu_sc as plsc

N = 8192
C_IN = 128
C_OUT = 256
K = 16
M = N // 4
EPS = 1e-5
BIGI = 2**30


_FR, _FC = 64, 128
_KT = 128
_CHR = 512
INF = float("inf")


def _geo_body(px_ref, py_ref, pz_ref, pxc_ref, pyc_ref, pzc_ref,
              pxs_ref, pys_ref, pzs_ref,
              npx_ref, npy_ref, npz_ref, knn_ref, d2_ref):
    i = pl.program_id(0)

    @pl.when(i == 0)
    def _fps():
        rows = lax.broadcasted_iota(jnp.int32, (_FR, _FC), 0)
        cols = lax.broadcasted_iota(jnp.int32, (_FR, _FC), 1)
        lin = rows * _FC + cols
        l128 = lax.broadcasted_iota(jnp.int32, (1, _FC), 1)
        px = px_ref[...]
        py = py_ref[...]
        pz = pz_ref[...]
        qx0 = pxs_ref[0]
        qy0 = pys_ref[0]
        qz0 = pzs_ref[0]
        zrow = jnp.zeros((1, _FC), jnp.float32)
        rax0 = jnp.where(l128 == 0, qx0, zrow)
        ray0 = jnp.where(l128 == 0, qy0, zrow)
        raz0 = jnp.where(l128 == 0, qz0, zrow)

        def step(j, carry):
            dists, qx, qy, qz, rax, ray, raz = carry
            dx = px - qx
            dy = py - qy
            dz = pz - qz
            d = dx * dx + dy * dy + dz * dz
            dists = jnp.minimum(dists, d)
            a = jnp.maximum(dists[0:32], dists[32:64])
            a = jnp.maximum(a[0:16], a[16:32])
            a = jnp.maximum(a[0:8], a[8:16])
            a = jnp.max(a, axis=0, keepdims=True)
            mx = jnp.max(a, axis=1, keepdims=True)
            cand = jnp.where(dists == mx, lin, BIGI)
            b = jnp.minimum(cand[0:32], cand[32:64])
            b = jnp.minimum(b[0:16], b[16:32])
            b = jnp.minimum(b[0:8], b[8:16])
            b = jnp.min(b, axis=0, keepdims=True)
            nxt = jnp.min(b, axis=1, keepdims=True)[0, 0]
            nx = pxs_ref[nxt]
            ny = pys_ref[nxt]
            nz = pzs_ref[nxt]
            r = j // _FC
            l = j % _FC
            rax = jnp.where(l128 == l, nx, rax)
            ray = jnp.where(l128 == l, ny, ray)
            raz = jnp.where(l128 == l, nz, raz)

            @pl.when(l == _FC - 1)
            def _():
                npx_ref[pl.ds(r, 1), :] = rax
                npy_ref[pl.ds(r, 1), :] = ray
                npz_ref[pl.ds(r, 1), :] = raz

            keep = l < _FC - 1
            rax = jnp.where(keep, rax, 0.0)
            ray = jnp.where(keep, ray, 0.0)
            raz = jnp.where(keep, raz, 0.0)
            return (dists, nx, ny, nz, rax, ray, raz)

        init = (jnp.full((_FR, _FC), 1e10, jnp.float32), qx0, qy0, qz0,
                rax0, ray0, raz0)
        lax.fori_loop(1, M, step, init)

    @pl.when(i > 0)
    def _knn():
        t = i - 1
        cx = npx_ref[pl.ds(t, 1), :]
        cy = npy_ref[pl.ds(t, 1), :]
        cz = npz_ref[pl.ds(t, 1), :]
        for c in range(N // _CHR):
            s = c * _CHR
            dx = pxc_ref[s:s + _CHR, :] - cx
            dy = pyc_ref[s:s + _CHR, :] - cy
            dz = pzc_ref[s:s + _CHR, :] - cz
            d2_ref[s:s + _CHR, :] = dx * dx + dy * dy + dz * dz
        iprev = jnp.full((1, _KT), -1, jnp.int32)
        for k in range(K):
            vbest = jnp.full((1, _KT), INF, jnp.float32)
            for c in range(N // _CHR):
                s = c * _CHR
                ri = lax.broadcasted_iota(jnp.int32, (_CHR, _KT), 0) + s
                blk = d2_ref[s:s + _CHR, :]
                if k > 0:
                    blk = jnp.where(ri == iprev, INF, blk)
                    d2_ref[s:s + _CHR, :] = blk
                vbest = jnp.minimum(vbest,
                                    jnp.min(blk, axis=0, keepdims=True))
            ibest = jnp.full((1, _KT), BIGI, jnp.int32)
            for c in range(N // _CHR):
                s = c * _CHR
                ri = lax.broadcasted_iota(jnp.int32, (_CHR, _KT), 0) + s
                blk = d2_ref[s:s + _CHR, :]
                cand = jnp.min(jnp.where(blk == vbest, ri, BIGI),
                               axis=0, keepdims=True)
                ibest = jnp.minimum(ibest, cand)
            knn_ref[k:k + 1, :] = ibest
            iprev = ibest


def _run_geo(p):
    px = p[:, 0].reshape(_FR, _FC)
    py = p[:, 1].reshape(_FR, _FC)
    pz = p[:, 2].reshape(_FR, _FC)
    cmap = lambda i: (0, 0)
    return pl.pallas_call(
        _geo_body,
        grid=(M // _KT + 1,),
        in_specs=[
            pl.BlockSpec((_FR, _FC), cmap),
            pl.BlockSpec((_FR, _FC), cmap),
            pl.BlockSpec((_FR, _FC), cmap),
            pl.BlockSpec((N, 1), cmap),
            pl.BlockSpec((N, 1), cmap),
            pl.BlockSpec((N, 1), cmap),
            pl.BlockSpec(memory_space=pltpu.SMEM),
            pl.BlockSpec(memory_space=pltpu.SMEM),
            pl.BlockSpec(memory_space=pltpu.SMEM),
        ],
        out_specs=[
            pl.BlockSpec((M // _FC, _FC), cmap),
            pl.BlockSpec((M // _FC, _FC), cmap),
            pl.BlockSpec((M // _FC, _FC), cmap),
            pl.BlockSpec((K, _KT), lambda i: (0, jnp.maximum(i - 1, 0))),
        ],
        out_shape=[
            jax.ShapeDtypeStruct((M // _FC, _FC), jnp.float32),
            jax.ShapeDtypeStruct((M // _FC, _FC), jnp.float32),
            jax.ShapeDtypeStruct((M // _FC, _FC), jnp.float32),
            jax.ShapeDtypeStruct((K, M), jnp.int32),
        ],
        scratch_shapes=[pltpu.VMEM((N, _KT), jnp.float32)],
    )(px, py, pz, p[:, 0:1], p[:, 1:2], p[:, 2:3],
      p[:, 0], p[:, 1], p[:, 2])


_MMB = 512
_NMM = N // _MMB


def _mm_body(x_ref, p_ref, w2_ref, wsp_ref, a_ref, g_ref, b_ref,
             y_ref, u_ref, h2_ref, acc_ref):
    ph = pl.program_id(0)
    i = pl.program_id(1)

    @pl.when(ph == 0)
    def _():
        xb = x_ref[...]
        h2 = jnp.dot(xb, w2_ref[...], preferred_element_type=jnp.float32)
        u = (jnp.dot(xb, wsp_ref[...], preferred_element_type=jnp.float32)
             + jnp.dot(p_ref[...], a_ref[...],
                       preferred_element_type=jnp.float32))
        h2_ref[pl.ds(i * _MMB, _MMB), :] = h2
        u_ref[...] = u
        s1 = jnp.sum(h2, axis=0, keepdims=True)
        s2 = jnp.sum(h2 * h2, axis=0, keepdims=True)

        @pl.when(i == 0)
        def _():
            acc_ref[0:1, :] = s1
            acc_ref[1:2, :] = s2

        @pl.when(i > 0)
        def _():
            acc_ref[0:1, :] = acc_ref[0:1, :] + s1
            acc_ref[1:2, :] = acc_ref[1:2, :] + s2

    @pl.when(ph == 1)
    def _():
        mu = acc_ref[0:1, :] / float(N)
        var = acc_ref[1:2, :] / float(N) - mu * mu
        y = (g_ref[...] * (h2_ref[pl.ds(i * _MMB, _MMB), :] - mu) / jnp.sqrt(var + EPS)
             + b_ref[...])
        y_ref[...] = jnp.maximum(y, 0.0)


def _run_mm(x, p, W2, Ws1p, A, g2, b2):
    return pl.pallas_call(
        _mm_body,
        grid=(2, _NMM),
        in_specs=[
            pl.BlockSpec((_MMB, C_IN), lambda p_, i: (i, 0)),
            pl.BlockSpec((_MMB, 3), lambda p_, i: (i, 0)),
            pl.BlockSpec((C_IN, C_OUT), lambda p_, i: (0, 0)),
            pl.BlockSpec((C_IN, C_IN), lambda p_, i: (0, 0)),
            pl.BlockSpec((3, C_IN), lambda p_, i: (0, 0)),
            pl.BlockSpec((1, C_OUT), lambda p_, i: (0, 0)),
            pl.BlockSpec((1, C_OUT), lambda p_, i: (0, 0)),
        ],
        out_specs=[
            pl.BlockSpec((_MMB, C_OUT),
                         lambda p_, i: (jnp.where(p_ == 1, i, 0), 0)),
            pl.BlockSpec((_MMB, C_IN),
                         lambda p_, i: (jnp.where(p_ == 0, i, _NMM - 1), 0)),
        ],
        out_shape=[
            jax.ShapeDtypeStruct((N, C_OUT), jnp.float32),
            jax.ShapeDtypeStruct((N, C_IN), jnp.float32),
        ],
        scratch_shapes=[
            pltpu.VMEM((N, C_OUT), jnp.float32),
            pltpu.VMEM((2, C_OUT), jnp.float32),
        ],
    )(x, p, W2, Ws1p, A, g2.reshape(1, C_OUT), b2.reshape(1, C_OUT))


_NC, _NS = 2, 16
_NW = _NC * _NS
_RW = (M * K) // _NW
_GC = 256


def _sc_gather_body(v_hbm, knn_hbm, vg_hbm, idx0, idx1, idx2, idx3,
                    buf0, buf1, sem0, sem1):
    wid = lax.axis_index("s") * _NC + lax.axis_index("c")
    base = wid * _RW
    idxs = (idx0, idx1, idx2, idx3)
    for c in range(_RW // _GC):
        pltpu.sync_copy(knn_hbm.at[pl.ds(base + c * _GC, _GC)], idxs[c])
    bufs = (buf0, buf1)
    sems = (sem0, sem1)
    cps = [None, None]
    cps[0] = pltpu.async_copy(v_hbm.at[idx0], buf0, sem0)
    for c in range(_RW // _GC):
        if c + 1 < _RW // _GC:
            cps[(c + 1) % 2] = pltpu.async_copy(
                v_hbm.at[idxs[c + 1]], bufs[(c + 1) % 2], sems[(c + 1) % 2])
        cps[c % 2].wait()
        pltpu.sync_copy(bufs[c % 2], vg_hbm.at[pl.ds(base + c * _GC, _GC)])


def _run_sc_gather(v, knn):
    mesh = plsc.VectorSubcoreMesh(core_axis_name="c", subcore_axis_name="s")
    f = pl.kernel(
        _sc_gather_body,
        out_type=jax.ShapeDtypeStruct((M * K, C_IN), jnp.float32),
        mesh=mesh,
        scratch_types=[
            pltpu.VMEM((_GC,), jnp.int32),
            pltpu.VMEM((_GC,), jnp.int32),
            pltpu.VMEM((_GC,), jnp.int32),
            pltpu.VMEM((_GC,), jnp.int32),
            pltpu.VMEM((_GC, C_IN), jnp.float32),
            pltpu.VMEM((_GC, C_IN), jnp.float32),
            pltpu.SemaphoreType.DMA,
            pltpu.SemaphoreType.DMA,
        ],
    )
    return f(v, knn.reshape(M * K))


_PB = 2048
_NPB = (M * K) // _PB


def _shr_body(vg_ref, np_ref, a_ref, g_ref, b_ref, w_ref, bs_ref,
              prob_ref, pre_ref, acc_ref):
    ph = pl.program_id(0)
    i = pl.program_id(1)
    nm = _PB // K
    r16 = lax.broadcasted_iota(jnp.int32, (_PB, nm), 0) // K
    c16 = lax.broadcasted_iota(jnp.int32, (_PB, nm), 1)
    e01 = jnp.where(r16 == c16, 1.0, 0.0)

    @pl.when(ph == 0)
    def _():
        w = jnp.dot(np_ref[...], a_ref[...],
                    preferred_element_type=jnp.float32)
        w_exp = jnp.dot(e01, w, preferred_element_type=jnp.float32)
        pre = vg_ref[...] - w_exp
        pre_ref[pl.ds(i * _PB, _PB), :] = pre
        s1 = jnp.sum(pre, axis=0, keepdims=True)
        s2 = jnp.sum(pre * pre, axis=0, keepdims=True)

        @pl.when(i == 0)
        def _():
            acc_ref[0:1, :] = s1
            acc_ref[1:2, :] = s2

        @pl.when(i > 0)
        def _():
            acc_ref[0:1, :] = acc_ref[0:1, :] + s1
            acc_ref[1:2, :] = acc_ref[1:2, :] + s2

    @pl.when(ph == 1)
    def _():
        n_rows = float(M * K)
        mu = acc_ref[0:1, :] / n_rows
        var = acc_ref[1:2, :] / n_rows - mu * mu
        h = (g_ref[...] * (pre_ref[pl.ds(i * _PB, _PB), :] - mu)
             / jnp.sqrt(var + EPS) + b_ref[...])
        h = jnp.maximum(h, 0.0)
        s = jnp.sum(h * w_ref[...], axis=1, keepdims=True) + bs_ref[0, 0]
        ex = jnp.exp(s)
        seg = jnp.dot(e01.T, ex, preferred_element_type=jnp.float32)
        den = jnp.dot(e01, seg, preferred_element_type=jnp.float32)
        prob_ref[...] = ex / den


def _run_shr(vg, n_p, A, gs1, bs1, Ws2, bs2):
    return pl.pallas_call(
        _shr_body,
        grid=(2, _NPB),
        in_specs=[
            pl.BlockSpec((_PB, C_IN), lambda p_, i: (i, 0)),
            pl.BlockSpec((_PB // K, 3), lambda p_, i: (i, 0)),
            pl.BlockSpec((3, C_IN), lambda p_, i: (0, 0)),
            pl.BlockSpec((1, C_IN), lambda p_, i: (0, 0)),
            pl.BlockSpec((1, C_IN), lambda p_, i: (0, 0)),
            pl.BlockSpec((1, C_IN), lambda p_, i: (0, 0)),
            pl.BlockSpec((1, 1), lambda p_, i: (0, 0),
                         memory_space=pltpu.SMEM),
        ],
        out_specs=pl.BlockSpec(
            (_PB, 1), lambda p_, i: (jnp.where(p_ == 1, i, 0), 0)),
        out_shape=jax.ShapeDtypeStruct((M * K, 1), jnp.float32),
        scratch_shapes=[
            pltpu.VMEM((M * K, C_IN), jnp.float32),
            pltpu.VMEM((2, C_IN), jnp.float32),
        ],
    )(vg, n_p, A, gs1.reshape(1, C_IN), bs1.reshape(1, C_IN),
      Ws2.reshape(1, C_IN), bs2.reshape(1, 1))


_MW = M // _NW
_GM = 4


def _sc_wsum_body(y_hbm, knn_hbm, prob_hbm, out_hbm, idxv, probv, buf0, buf1,
                  outb, sem0, sem1):
    wid = lax.axis_index("s") * _NC + lax.axis_index("c")
    base = wid * _MW * K
    pltpu.sync_copy(knn_hbm.at[pl.ds(base, _MW * K)], idxv)
    pltpu.sync_copy(prob_hbm.at[pl.ds(base, _MW * K)], probv)
    bufs = (buf0, buf1)
    sems = (sem0, sem1)
    dnums = lax.GatherDimensionNumbers(
        offset_dims=(), collapsed_slice_dims=(0,), start_index_map=(0,))

    def one_m(ml, buf):
        pm = probv[pl.ds(ml * K, K)]
        accs = [jnp.zeros((16,), jnp.float32) for _ in range(C_OUT // 16)]
        for k in range(K):
            ik = jnp.zeros((K, 1), jnp.int32) + k
            pk = lax.gather(pm, ik, dnums, (1,),
                            mode=lax.GatherScatterMode.PROMISE_IN_BOUNDS)
            for r in range(C_OUT // 16):
                accs[r] = accs[r] + pk * buf[k, pl.ds(r * 16, 16)]
        for r in range(C_OUT // 16):
            outb[ml, pl.ds(r * 16, 16)] = accs[r]

    def g_body(g, _):
        cps = []
        for j in range(2):
            iv = idxv[pl.ds((g * 2 + j) * K, K)]
            cps.append(pltpu.async_copy(y_hbm.at[iv], bufs[j], sems[j]))
        for j in range(2):
            cps[j].wait()
            one_m(g * 2 + j, bufs[j])
        return 0

    lax.fori_loop(0, _MW // 2, g_body, 0)
    pltpu.sync_copy(outb, out_hbm.at[pl.ds(wid * _MW, _MW)])


def _run_sc_wsum(y, knn, prob):
    mesh = plsc.VectorSubcoreMesh(core_axis_name="c", subcore_axis_name="s")
    f = pl.kernel(
        _sc_wsum_body,
        out_type=jax.ShapeDtypeStruct((M, C_OUT), jnp.float32),
        mesh=mesh,
        scratch_types=[
            pltpu.VMEM((_MW * K,), jnp.int32),
            pltpu.VMEM((_MW * K,), jnp.float32),
            pltpu.VMEM((K, C_OUT), jnp.float32),
            pltpu.VMEM((K, C_OUT), jnp.float32),
            pltpu.VMEM((_MW, C_OUT), jnp.float32),
            pltpu.SemaphoreType.DMA,
            pltpu.SemaphoreType.DMA,
        ],
    )
    return f(y, knn.reshape(M * K), prob.reshape(M * K))


def kernel(p, x, o, W2, g2, b2, Ws1, gs1, bs1, Ws2, bs2):
    npx, npy, npz, knnT = _run_geo(p)
    n_p = jnp.stack([npx.reshape(M), npy.reshape(M), npz.reshape(M)], axis=1)
    knn = knnT.T.reshape(M, K)
    y, v = _run_mm(x, p, W2, Ws1[3:], Ws1[:3], g2, b2)
    vg = _run_sc_gather(v, knn)
    prob = _run_shr(vg, n_p, Ws1[:3], gs1, bs1, Ws2, bs2)
    y_out = _run_sc_wsum(y, knn, prob)
    n_o = jnp.array([M], dtype=jnp.int32)
    return (n_p, y_out, n_o)
